# baseline (speedup 1.0000x reference)
import functools

import jax
import jax.numpy as jnp
from jax import lax
from jax.experimental import pallas as pl
from jax.experimental.pallas import tpu as pltpu
from jax.experimental.pallas import tpu_sc as plsc

_VOCAB = 29599
_GLOVE = 50
_DM = 128
_B = 16384
_L = 50

_NC = 2
_NS = 16
_NW = _NC * _NS
_PIECES = 4
_BPP = _B // _PIECES
_BPW = _BPP // _NW
_IPAD = 128
_NB = 4
_NGRP = _BPW // _NB

_PROJ_BLK = 1024


def _proj_body(tbl_ref, wt_ref, b_ref, out_ref):
    out_ref[...] = (
        jnp.dot(tbl_ref[...], wt_ref[...], preferred_element_type=jnp.float32)
        + b_ref[...]
    )


def _project_table(emb_table, W, b):
    v = emb_table.shape[0]
    vp = ((v + _PROJ_BLK - 1) // _PROJ_BLK) * _PROJ_BLK
    tbl = jnp.pad(emb_table, ((0, vp - v), (0, 0)))
    wt = W.T
    return pl.pallas_call(
        _proj_body,
        grid=(vp // _PROJ_BLK,),
        in_specs=[
            pl.BlockSpec((_PROJ_BLK, _GLOVE), lambda i: (i, 0)),
            pl.BlockSpec((_GLOVE, _DM), lambda i: (0, 0)),
            pl.BlockSpec((1, _DM), lambda i: (0, 0)),
        ],
        out_specs=pl.BlockSpec((_PROJ_BLK, _DM), lambda i: (i, 0)),
        out_shape=jax.ShapeDtypeStruct((vp, _DM), jnp.float32),
    )(tbl, wt, b.reshape(1, _DM))


def _sc_gather(p_tab, idx3):
    mesh = plsc.VectorSubcoreMesh(core_axis_name="c", subcore_axis_name="s")

    @functools.partial(
        pl.kernel,
        mesh=mesh,
        compiler_params=pltpu.CompilerParams(use_tc_tiling_on_sc=True),
        out_type=jax.ShapeDtypeStruct((_BPP, _L, _DM), jnp.float32),
        scratch_types=[
            pltpu.VMEM((_BPW, _IPAD), jnp.int32),
            pltpu.VMEM((_NB, _L, _DM), jnp.float32),
            pltpu.SemaphoreType.DMA((_NB,)),
            pltpu.SemaphoreType.DMA((_NB,)),
        ],
    )
    def k(p_hbm, idx_hbm, out_hbm, idx_v, bufs, gsem, wsem):
        wid = lax.axis_index("s") * _NC + lax.axis_index("c")
        pltpu.sync_copy(idx_hbm.at[wid], idx_v)
        base = wid * _BPW

        def start_gather(j, s):
            pltpu.async_copy(
                p_hbm.at[idx_v.at[j, pl.ds(0, _L)]], bufs.at[s], gsem.at[s]
            )

        def wait_gather(j, s):
            pltpu.make_async_copy(
                p_hbm.at[idx_v.at[j, pl.ds(0, _L)]], bufs.at[s], gsem.at[s]
            ).wait()

        def out_slice(j):
            return out_hbm.at[base + j]

        def start_write(j, s):
            pltpu.async_copy(bufs.at[s], out_slice(j), wsem.at[s])

        def wait_write(j, s):
            pltpu.make_async_copy(bufs.at[s], out_slice(j), wsem.at[s]).wait()

        for s in range(_NB):
            start_gather(s, s)

        def group_body(g, _):
            for s in range(_NB):
                j = g * _NB + s
                wait_gather(j, s)
                start_write(j, s)
            for s in range(_NB):
                j = g * _NB + s
                wait_write(j, s)
                start_gather(j + _NB, s)
            return 0

        lax.fori_loop(0, _NGRP - 1, group_body, 0)

        for s in range(_NB):
            j = (_NGRP - 1) * _NB + s
            wait_gather(j, s)
            start_write(j, s)
        for s in range(_NB):
            j = (_NGRP - 1) * _NB + s
            wait_write(j, s)

    return k(p_tab, idx3)


def kernel(x, emb_table, W, b):
    p_tab = _project_table(emb_table, W, b)
    xi = jnp.pad(x.astype(jnp.int32), ((0, 0), (0, _IPAD - _L)))
    idx4 = xi.reshape(_PIECES, _NW, _BPW, _IPAD)
    pieces = [_sc_gather(p_tab, idx4[k]) for k in range(_PIECES)]
    return jnp.concatenate(pieces, axis=0)


# speedup vs baseline: 1.7417x; 1.7417x over previous
import functools

import jax
import jax.numpy as jnp
from jax import lax
from jax.experimental import pallas as pl
from jax.experimental.pallas import tpu as pltpu
from jax.experimental.pallas import tpu_sc as plsc

_VOCAB = 29599
_GLOVE = 50
_DM = 128
_B = 16384
_L = 50

_NC = 2
_NS = 16
_NW = _NC * _NS
_PIECES = 1
_BPP = _B // _PIECES
_BPW = _BPP // _NW
_IPAD = 128
_NB = 8
_NGRP = _BPW // _NB

_PROJ_BLK = 1024


def _proj_body(tbl_ref, wt_ref, b_ref, out_ref):
    out_ref[...] = (
        jnp.dot(tbl_ref[...], wt_ref[...], preferred_element_type=jnp.float32)
        + b_ref[...]
    )


def _project_table(emb_table, W, b):
    v = emb_table.shape[0]
    vp = ((v + _PROJ_BLK - 1) // _PROJ_BLK) * _PROJ_BLK
    tbl = jnp.pad(emb_table, ((0, vp - v), (0, 0)))
    wt = W.T
    return pl.pallas_call(
        _proj_body,
        grid=(vp // _PROJ_BLK,),
        in_specs=[
            pl.BlockSpec((_PROJ_BLK, _GLOVE), lambda i: (i, 0)),
            pl.BlockSpec((_GLOVE, _DM), lambda i: (0, 0)),
            pl.BlockSpec((1, _DM), lambda i: (0, 0)),
        ],
        out_specs=pl.BlockSpec((_PROJ_BLK, _DM), lambda i: (i, 0)),
        out_shape=jax.ShapeDtypeStruct((vp, _DM), jnp.float32),
    )(tbl, wt, b.reshape(1, _DM))


def _sc_gather(p_tab, idx3):
    mesh = plsc.VectorSubcoreMesh(core_axis_name="c", subcore_axis_name="s")

    @functools.partial(
        pl.kernel,
        mesh=mesh,
        compiler_params=pltpu.CompilerParams(use_tc_tiling_on_sc=True),
        out_type=jax.ShapeDtypeStruct((_BPP, _L, _DM), jnp.float32),
        scratch_types=[
            pltpu.VMEM((_BPW, _IPAD), jnp.int32),
            pltpu.VMEM((_NB, _L, _DM), jnp.float32),
            pltpu.SemaphoreType.DMA((_NB,)),
            pltpu.SemaphoreType.DMA((_NB,)),
        ],
    )
    def k(p_hbm, idx_hbm, out_hbm, idx_v, bufs, gsem, wsem):
        wid = lax.axis_index("s") * _NC + lax.axis_index("c")
        pltpu.sync_copy(idx_hbm.at[wid], idx_v)
        base = wid * _BPW

        def start_gather(j, s):
            pltpu.async_copy(
                p_hbm.at[idx_v.at[j, pl.ds(0, _L)]], bufs.at[s], gsem.at[s]
            )

        def wait_gather(j, s):
            pltpu.make_async_copy(
                p_hbm.at[idx_v.at[j, pl.ds(0, _L)]], bufs.at[s], gsem.at[s]
            ).wait()

        def out_slice(j):
            return out_hbm.at[base + j]

        def start_write(j, s):
            pltpu.async_copy(bufs.at[s], out_slice(j), wsem.at[s])

        def wait_write(j, s):
            pltpu.make_async_copy(bufs.at[s], out_slice(j), wsem.at[s]).wait()

        for s in range(_NB):
            start_gather(s, s)

        def group_body(g, _):
            for s in range(_NB):
                j = g * _NB + s
                wait_gather(j, s)
                start_write(j, s)
            for s in range(_NB):
                j = g * _NB + s
                wait_write(j, s)
                start_gather(j + _NB, s)
            return 0

        lax.fori_loop(0, _NGRP - 1, group_body, 0)

        for s in range(_NB):
            j = (_NGRP - 1) * _NB + s
            wait_gather(j, s)
            start_write(j, s)
        for s in range(_NB):
            j = (_NGRP - 1) * _NB + s
            wait_write(j, s)

    return k(p_tab, idx3)


def kernel(x, emb_table, W, b):
    p_tab = _project_table(emb_table, W, b)
    xi = jnp.pad(x.astype(jnp.int32), ((0, 0), (0, _IPAD - _L)))
    idx4 = xi.reshape(_PIECES, _NW, _BPW, _IPAD)
    pieces = [_sc_gather(p_tab, idx4[k]) for k in range(_PIECES)]
    return jnp.concatenate(pieces, axis=0)


# speedup vs baseline: 1.7481x; 1.0036x over previous
import functools

import jax
import jax.numpy as jnp
from jax import lax
from jax.experimental import pallas as pl
from jax.experimental.pallas import tpu as pltpu
from jax.experimental.pallas import tpu_sc as plsc

_VOCAB = 29599
_GLOVE = 50
_DM = 128
_B = 16384
_L = 50

_NC = 2
_NS = 16
_NW = _NC * _NS
_BPW = _B // _NW
_TPW = _BPW * _L
_GB = 4
_GROWS = _GB * _L
_NB = 4
_NGRP = _BPW // _GB
_CHUNKS = ((0, 128), (128, 72))

_PROJ_BLK = 1024


def _proj_body(tbl_ref, wt_ref, b_ref, out_ref):
    out_ref[...] = (
        jnp.dot(tbl_ref[...], wt_ref[...], preferred_element_type=jnp.float32)
        + b_ref[...]
    )


def _project_table(emb_table, W, b):
    v = emb_table.shape[0]
    vp = ((v + _PROJ_BLK - 1) // _PROJ_BLK) * _PROJ_BLK
    tbl = jnp.pad(emb_table, ((0, vp - v), (0, 0)))
    wt = W.T
    return pl.pallas_call(
        _proj_body,
        grid=(vp // _PROJ_BLK,),
        in_specs=[
            pl.BlockSpec((_PROJ_BLK, _GLOVE), lambda i: (i, 0)),
            pl.BlockSpec((_GLOVE, _DM), lambda i: (0, 0)),
            pl.BlockSpec((1, _DM), lambda i: (0, 0)),
        ],
        out_specs=pl.BlockSpec((_PROJ_BLK, _DM), lambda i: (i, 0)),
        out_shape=jax.ShapeDtypeStruct((vp, _DM), jnp.float32),
    )(tbl, wt, b.reshape(1, _DM))


def _sc_gather(p_tab, idx2):
    mesh = plsc.VectorSubcoreMesh(core_axis_name="c", subcore_axis_name="s")

    @functools.partial(
        pl.kernel,
        mesh=mesh,
        out_type=jax.ShapeDtypeStruct((_B, _L, _DM), jnp.float32),
        scratch_types=[
            pltpu.VMEM((_TPW,), jnp.int32),
            pltpu.VMEM((_NB, _GROWS, _DM), jnp.float32),
            pltpu.SemaphoreType.DMA((_NB,)),
            pltpu.SemaphoreType.DMA((_NB,)),
        ],
    )
    def k(p_hbm, idx_hbm, out_hbm, idx_v, bufs, gsem, wsem):
        wid = lax.axis_index("s") * _NC + lax.axis_index("c")
        pltpu.sync_copy(idx_hbm.at[wid], idx_v)
        base = wid * _BPW

        def start_gather(g, s):
            for off, ln in _CHUNKS:
                pltpu.async_copy(
                    p_hbm.at[idx_v.at[pl.ds(g * _GROWS + off, ln)]],
                    bufs.at[s, pl.ds(off, ln)],
                    gsem.at[s],
                )

        def wait_gather(g, s):
            for off, ln in _CHUNKS:
                pltpu.make_async_copy(
                    p_hbm.at[idx_v.at[pl.ds(g * _GROWS + off, ln)]],
                    bufs.at[s, pl.ds(off, ln)],
                    gsem.at[s],
                ).wait()

        def start_write(g, s):
            for i in range(_GB):
                pltpu.async_copy(
                    bufs.at[s, pl.ds(i * _L, _L)],
                    out_hbm.at[base + g * _GB + i],
                    wsem.at[s],
                )

        def wait_write(g, s):
            for i in range(_GB):
                pltpu.make_async_copy(
                    bufs.at[s, pl.ds(i * _L, _L)],
                    out_hbm.at[base + g * _GB + i],
                    wsem.at[s],
                ).wait()

        for s in range(_NB):
            start_gather(s, s)

        def group_body(g, _):
            for s in range(_NB):
                j = g * _NB + s
                wait_gather(j, s)
                start_write(j, s)
            for s in range(_NB):
                j = g * _NB + s
                wait_write(j, s)
                start_gather(j + _NB, s)
            return 0

        lax.fori_loop(0, _NGRP // _NB - 1, group_body, 0)

        for s in range(_NB):
            j = (_NGRP // _NB - 1) * _NB + s
            wait_gather(j, s)
            start_write(j, s)
        for s in range(_NB):
            j = (_NGRP // _NB - 1) * _NB + s
            wait_write(j, s)

    return k(p_tab, idx2)


def kernel(x, emb_table, W, b):
    p_tab = _project_table(emb_table, W, b)
    idx2 = x.astype(jnp.int32).reshape(_NW, _TPW)
    return _sc_gather(p_tab, idx2)
